# baseline (speedup 1.0000x reference)
import functools

import jax
import jax.numpy as jnp
from jax import lax
from jax.experimental import pallas as pl
from jax.experimental.pallas import tpu as pltpu
from jax.experimental.pallas import tpu_sc as plsc

N = 10000
IN_CH = 256
OUT_CH = 256
H = 4
E = 160000

BLK = 400
GRID = N // BLK

NPASS = 9
NA = 10240
SA = NA // 16

K = 112
ET = 10080
EP = 16 * ET
CHF = ET // K
ETG = 5152
EG = 32 * ETG
CHG = ETG // K


def _tc1_body(x_ref, wv_ref, wa_ref, p_ref):
    xb = x_ref[...]
    a = jnp.dot(xb, wa_ref[...], preferred_element_type=jnp.float32)
    g = jnp.exp(a)
    vals = jnp.dot(xb, wv_ref[...], preferred_element_type=jnp.float32)
    for p in range(8):
        for h in range(H):
            c0 = h * OUT_CH + p * 32
            p_ref[p, :, h * 32:(h + 1) * 32] = vals[:, c0:c0 + 32] * g[:, h:h + 1]
    p_ref[8, :, 0:H] = g
    p_ref[8, :, H:128] = jnp.zeros((BLK, 128 - H), jnp.float32)


def _tc1(x, wv, wa):
    return pl.pallas_call(
        _tc1_body,
        grid=(GRID,),
        in_specs=[
            pl.BlockSpec((BLK, IN_CH), lambda i: (i, 0)),
            pl.BlockSpec((IN_CH, H * OUT_CH), lambda i: (0, 0)),
            pl.BlockSpec((IN_CH, H), lambda i: (0, 0)),
        ],
        out_specs=pl.BlockSpec((NPASS, BLK, 128), lambda i: (0, i, 0)),
        out_shape=jax.ShapeDtypeStruct((NPASS, N, 128), jnp.float32),
    )(x, wv, wa)


def _tc2_body(acc_ref, x_ref, wr_ref, br_ref, out_ref):
    s = acc_ref[8, :, 0:H] + acc_ref[9, :, 0:H]
    sinv = jnp.where(s > 0.0, 0.25 / s, 0.0)
    xb = x_ref[...]
    root = jnp.dot(xb, wr_ref[...], preferred_element_type=jnp.float32) + br_ref[...]
    cols = []
    for p in range(8):
        accp = acc_ref[p]
        seg = accp[:, 0:32] * sinv[:, 0:1]
        for h in range(1, H):
            seg = seg + accp[:, h * 32:(h + 1) * 32] * sinv[:, h:h + 1]
        cols.append(seg)
    out_ref[...] = jnp.concatenate(cols, axis=1) + root


def _tc2(acc, x, wr, br):
    return pl.pallas_call(
        _tc2_body,
        grid=(GRID,),
        in_specs=[
            pl.BlockSpec((NPASS + 1, BLK, 128), lambda i: (0, i, 0)),
            pl.BlockSpec((BLK, IN_CH), lambda i: (i, 0)),
            pl.BlockSpec((IN_CH, OUT_CH), lambda i: (0, 0)),
            pl.BlockSpec((1, OUT_CH), lambda i: (0, 0)),
        ],
        out_specs=pl.BlockSpec((BLK, OUT_CH), lambda i: (i, 0)),
        out_shape=jax.ShapeDtypeStruct((N, OUT_CH), jnp.float32),
    )(acc, x, wr, br)


_MESH = plsc.VectorSubcoreMesh(core_axis_name="c", subcore_axis_name="s")


@functools.partial(
    pl.kernel,
    mesh=_MESH,
    out_type=jax.ShapeDtypeStruct(((NPASS + 1) * NA, 128), jnp.float32),
    scratch_types=[
        pltpu.VMEM((ET,), jnp.int32),
        pltpu.VMEM((ET,), jnp.int32),
        pltpu.VMEM((K, 128), jnp.float32),
        pltpu.VMEM((K, 128), jnp.float32),
        pltpu.SemaphoreType.DMA,
        pltpu.SemaphoreType.DMA,
        pltpu.SemaphoreType.DMA,
        pltpu.SemaphoreType.DMA,
        pltpu.VMEM_SHARED((NA, 128), jnp.float32),
    ],
)
def _sc2(p_hbm, rowsf_hbm, col_hbm, rowsg_hbm, colg_hbm, z128_hbm,
         acc_hbm, row_v, col_v, b0, b1, g0, g1, s0, s1, acc):
    cid = lax.axis_index("c")
    sid = lax.axis_index("s")
    bufs = [b0, b1]
    gsems = [g0, g1]
    ssems = [s0, s1]

    def gissue(c, k):
        pltpu.async_copy(p_hbm.at[row_v.at[pl.ds(c * K, K)]], bufs[k], gsems[k])

    def gwait(k):
        pltpu.make_async_copy(
            p_hbm.at[row_v.at[pl.ds(0, K)]], bufs[k], gsems[k]
        ).wait()

    def sissue(c, k):
        for i in range(K // 16):
            cvec = col_v[pl.ds(c * K + i * 16, 16)]
            pltpu.async_copy(
                bufs[k].at[pl.ds(i * 16, 16)], acc.at[cvec], ssems[k], add=True
            )

    def sdrain(k):
        pltpu.make_async_copy(
            p_hbm.at[row_v.at[pl.ds(0, K)]], bufs[k], ssems[k]
        ).wait()

    def run_pass(nchunk, obase):
        pltpu.sync_copy(z128_hbm, acc.at[pl.ds(sid * SA, SA)])
        plsc.subcore_barrier()
        gissue(0, 0)
        gissue(1, 1)

        def body(j, carry):
            c0 = 2 * j
            gwait(0)
            sissue(c0, 0)
            gwait(1)
            sdrain(0)

            @pl.when(c0 + 2 <= nchunk - 1)
            def _():
                gissue(c0 + 2, 0)

            sissue(c0 + 1, 1)
            sdrain(1)

            @pl.when(c0 + 3 <= nchunk - 1)
            def _():
                gissue(c0 + 3, 1)

            return carry

        lax.fori_loop(0, nchunk // 2, body, 0)
        plsc.subcore_barrier()
        pltpu.sync_copy(
            acc.at[pl.ds(sid * SA, SA)],
            acc_hbm.at[pl.ds(obase + sid * SA, SA)],
        )
        plsc.subcore_barrier()

    pltpu.sync_copy(col_hbm.at[pl.ds(sid * ET, ET)], col_v)
    for p in range(4):
        pass_id = cid * 4 + p
        pltpu.sync_copy(
            rowsf_hbm.at[pl.ds(pass_id * EP + sid * ET, ET)], row_v
        )
        run_pass(CHF, pass_id * NA)

    wbase = (cid * 16 + sid) * ETG
    pltpu.sync_copy(rowsg_hbm.at[pl.ds(wbase, ETG)], row_v.at[pl.ds(0, ETG)])
    pltpu.sync_copy(colg_hbm.at[pl.ds(wbase, ETG)], col_v.at[pl.ds(0, ETG)])
    run_pass(CHG, (8 + cid) * NA)


@jax.jit
def kernel(x, edge_index, W_value, W_assign, b_assign, W_root, b_root):
    del b_assign
    row = edge_index[0].astype(jnp.int32)
    col = edge_index[1].astype(jnp.int32)

    P = _tc1(x, W_value, W_assign)

    padr = jnp.arange(EP - E, dtype=jnp.int32) % N
    padc = N + jnp.arange(EP - E, dtype=jnp.int32) % (NA - N)
    rowp = jnp.concatenate([row, padr])
    colp = jnp.concatenate([col, padc])
    rowsf = (rowp[None, :] + (jnp.arange(8, dtype=jnp.int32) * N)[:, None]).reshape(-1)
    padrg = jnp.arange(EG - E, dtype=jnp.int32) % N
    padcg = N + jnp.arange(EG - E, dtype=jnp.int32) % (NA - N)
    rowsg = jnp.concatenate([row + 8 * N, padrg + 8 * N])
    colg = jnp.concatenate([col, padcg])

    accU = _sc2(
        P.reshape(NPASS * N, 128), rowsf, colp, rowsg, colg,
        jnp.zeros((SA, 128), jnp.float32),
    )

    return _tc2(
        accU.reshape(NPASS + 1, NA, 128), x, W_root, b_root.reshape(1, OUT_CH)
    )


# speedup vs baseline: 1.0277x; 1.0277x over previous
import functools

import jax
import jax.numpy as jnp
from jax import lax
from jax.experimental import pallas as pl
from jax.experimental.pallas import tpu as pltpu
from jax.experimental.pallas import tpu_sc as plsc

N = 10000
IN_CH = 256
OUT_CH = 256
H = 4
E = 160000

BLK = 400
GRID = N // BLK

NPASS = 9
NA = 10240
SA = NA // 16

K = 112
ET = 10080
EP = 16 * ET
CHF = ET // K
ETG = 5152
EG = 32 * ETG
CHG = ETG // K


def _tc1_body(x_ref, wv_ref, wa_ref, p_ref):
    xb = x_ref[...]
    a = jnp.dot(xb, wa_ref[...], preferred_element_type=jnp.float32)
    g = jnp.exp(a)
    vals = jnp.dot(xb, wv_ref[...], preferred_element_type=jnp.float32)
    for p in range(8):
        for h in range(H):
            c0 = h * OUT_CH + p * 32
            p_ref[p, :, h * 32:(h + 1) * 32] = vals[:, c0:c0 + 32] * g[:, h:h + 1]
    p_ref[8, :, 0:H] = g
    p_ref[8, :, H:128] = jnp.zeros((BLK, 128 - H), jnp.float32)


def _tc1(x, wv, wa):
    return pl.pallas_call(
        _tc1_body,
        grid=(GRID,),
        in_specs=[
            pl.BlockSpec((BLK, IN_CH), lambda i: (i, 0)),
            pl.BlockSpec((IN_CH, H * OUT_CH), lambda i: (0, 0)),
            pl.BlockSpec((IN_CH, H), lambda i: (0, 0)),
        ],
        out_specs=pl.BlockSpec((NPASS, BLK, 128), lambda i: (0, i, 0)),
        out_shape=jax.ShapeDtypeStruct((NPASS, N, 128), jnp.float32),
    )(x, wv, wa)


def _tc2_body(acc_ref, x_ref, wr_ref, br_ref, out_ref):
    s = acc_ref[8, :, 0:H] + acc_ref[9, :, 0:H]
    sinv = jnp.where(s > 0.0, 0.25 / s, 0.0)
    xb = x_ref[...]
    root = jnp.dot(xb, wr_ref[...], preferred_element_type=jnp.float32) + br_ref[...]
    cols = []
    for p in range(8):
        accp = acc_ref[p]
        seg = accp[:, 0:32] * sinv[:, 0:1]
        for h in range(1, H):
            seg = seg + accp[:, h * 32:(h + 1) * 32] * sinv[:, h:h + 1]
        cols.append(seg)
    out_ref[...] = jnp.concatenate(cols, axis=1) + root


def _tc2(acc, x, wr, br):
    return pl.pallas_call(
        _tc2_body,
        grid=(GRID,),
        in_specs=[
            pl.BlockSpec((NPASS + 1, BLK, 128), lambda i: (0, i, 0)),
            pl.BlockSpec((BLK, IN_CH), lambda i: (i, 0)),
            pl.BlockSpec((IN_CH, OUT_CH), lambda i: (0, 0)),
            pl.BlockSpec((1, OUT_CH), lambda i: (0, 0)),
        ],
        out_specs=pl.BlockSpec((BLK, OUT_CH), lambda i: (i, 0)),
        out_shape=jax.ShapeDtypeStruct((N, OUT_CH), jnp.float32),
    )(acc, x, wr, br)


_MESH = plsc.VectorSubcoreMesh(core_axis_name="c", subcore_axis_name="s")


@functools.partial(
    pl.kernel,
    mesh=_MESH,
    out_type=jax.ShapeDtypeStruct(((NPASS + 1) * NA, 128), jnp.float32),
    scratch_types=[
        pltpu.VMEM((ET,), jnp.int32),
        pltpu.VMEM((ET,), jnp.int32),
        pltpu.VMEM((K, 128), jnp.float32),
        pltpu.VMEM((K, 128), jnp.float32),
        pltpu.SemaphoreType.DMA,
        pltpu.SemaphoreType.DMA,
        pltpu.SemaphoreType.DMA,
        pltpu.SemaphoreType.DMA,
        pltpu.VMEM_SHARED((NA, 128), jnp.float32),
    ],
)
def _sc2(p_hbm, rowsf_hbm, col_hbm, rowsg_hbm, colg_hbm, z128_hbm,
         acc_hbm, row_v, col_v, b0, b1, g0, g1, s0, s1, acc):
    cid = lax.axis_index("c")
    sid = lax.axis_index("s")
    bufs = [b0, b1]
    gsems = [g0, g1]
    ssems = [s0, s1]

    def gissue(c, k):
        pltpu.async_copy(p_hbm.at[row_v.at[pl.ds(c * K, K)]], bufs[k], gsems[k])

    def gwait(k):
        pltpu.make_async_copy(
            p_hbm.at[row_v.at[pl.ds(0, K)]], bufs[k], gsems[k]
        ).wait()

    def scatter(c, k):
        for i in range(K // 16):
            cvec = col_v[pl.ds(c * K + i * 16, 16)]
            pltpu.async_copy(
                bufs[k].at[pl.ds(i * 16, 16)], acc.at[cvec], ssems[k], add=True
            )
        pltpu.make_async_copy(
            p_hbm.at[row_v.at[pl.ds(0, K)]], bufs[k], ssems[k]
        ).wait()

    def run_pass(nchunk, obase):
        pltpu.sync_copy(z128_hbm, acc.at[pl.ds(sid * SA, SA)])
        plsc.subcore_barrier()
        gissue(0, 0)
        gissue(1, 1)

        def body(j, carry):
            c0 = 2 * j
            gwait(0)
            scatter(c0, 0)

            @pl.when(c0 + 2 <= nchunk - 1)
            def _():
                gissue(c0 + 2, 0)

            gwait(1)
            scatter(c0 + 1, 1)

            @pl.when(c0 + 3 <= nchunk - 1)
            def _():
                gissue(c0 + 3, 1)

            return carry

        lax.fori_loop(0, nchunk // 2, body, 0)
        plsc.subcore_barrier()
        pltpu.sync_copy(
            acc.at[pl.ds(sid * SA, SA)],
            acc_hbm.at[pl.ds(obase + sid * SA, SA)],
        )
        plsc.subcore_barrier()

    pltpu.sync_copy(col_hbm.at[pl.ds(sid * ET, ET)], col_v)
    for p in range(4):
        pass_id = cid * 4 + p
        pltpu.sync_copy(
            rowsf_hbm.at[pl.ds(pass_id * EP + sid * ET, ET)], row_v
        )
        run_pass(CHF, pass_id * NA)

    wbase = (cid * 16 + sid) * ETG
    pltpu.sync_copy(rowsg_hbm.at[pl.ds(wbase, ETG)], row_v.at[pl.ds(0, ETG)])
    pltpu.sync_copy(colg_hbm.at[pl.ds(wbase, ETG)], col_v.at[pl.ds(0, ETG)])
    run_pass(CHG, (8 + cid) * NA)


@jax.jit
def kernel(x, edge_index, W_value, W_assign, b_assign, W_root, b_root):
    del b_assign
    row = edge_index[0].astype(jnp.int32)
    col = edge_index[1].astype(jnp.int32)

    P = _tc1(x, W_value, W_assign)

    padr = jnp.arange(EP - E, dtype=jnp.int32) % N
    padc = N + jnp.arange(EP - E, dtype=jnp.int32) % (NA - N)
    rowp = jnp.concatenate([row, padr])
    colp = jnp.concatenate([col, padc])
    rowsf = (rowp[None, :] + (jnp.arange(8, dtype=jnp.int32) * N)[:, None]).reshape(-1)
    padrg = jnp.arange(EG - E, dtype=jnp.int32) % N
    padcg = N + jnp.arange(EG - E, dtype=jnp.int32) % (NA - N)
    rowsg = jnp.concatenate([row + 8 * N, padrg + 8 * N])
    colg = jnp.concatenate([col, padcg])

    accU = _sc2(
        P.reshape(NPASS * N, 128), rowsf, colp, rowsg, colg,
        jnp.zeros((SA, 128), jnp.float32),
    )

    return _tc2(
        accU.reshape(NPASS + 1, NA, 128), x, W_root, b_root.reshape(1, OUT_CH)
    )
